# baseline (speedup 1.0000x reference)
import functools

import jax
import jax.numpy as jnp
from jax import lax
from jax.experimental import pallas as pl
from jax.experimental.pallas import tpu as pltpu
from jax.experimental.pallas import tpu_sc as plsc

_N = 10000
_E = 320000
_D = 128
_OUT = 40

_NC = 2
_NS = 16
_NW = _NC * _NS
_CHUNK = 128
_STEPS = 82
_EP = _NW * _STEPS * _CHUNK
_NP = 10240
_ACHUNK = 128
_ABS = 28
_ANBLK = 3
_ASTEPS = _ABS * _ANBLK
_AEP = _NW * _ASTEPS * _ACHUNK
_NPA = 10112
_RPTA = _NPA // _NS

_BLK = 512
_GRID = _NP // _BLK

_sc_mesh = plsc.VectorSubcoreMesh(
    core_axis_name="c", subcore_axis_name="s", num_cores=_NC, num_subcores=_NS)


@functools.partial(
    pl.kernel,
    out_type=jax.ShapeDtypeStruct((_NW * _NP,), jnp.float32),
    mesh=_sc_mesh,
    scratch_types=[
        pltpu.VMEM((_STEPS * _CHUNK,), jnp.int32),
        pltpu.VMEM((_NP,), jnp.float32),
    ],
    compiler_params=pltpu.CompilerParams(needs_layout_passes=False),
)
def _deg_sc(dsts_hbm, zeros_hbm, out_hbm, dst_v, deg_l):
    c = lax.axis_index("c")
    s = lax.axis_index("s")
    w = c * _NS + s
    pltpu.sync_copy(zeros_hbm, deg_l)
    pltpu.sync_copy(dsts_hbm.at[w], dst_v)
    ones = jnp.ones((16,), jnp.float32)

    @pl.loop(0, _STEPS * _CHUNK // 16)
    def _(j):
        idx = dst_v[pl.ds(j * 16, 16)]
        plsc.addupdate_scatter(deg_l, [idx], ones)

    pltpu.sync_copy(deg_l, out_hbm.at[pl.ds(w * _NP, _NP)])


@functools.partial(
    pl.kernel,
    out_type=jax.ShapeDtypeStruct((_NC * _NPA, _D), jnp.float32),
    mesh=_sc_mesh,
    scratch_types=[
        pltpu.VMEM((2, _ABS + 2, _ACHUNK), jnp.int32),
        pltpu.VMEM((2, _ABS + 2, _ACHUNK), jnp.int32),
        pltpu.VMEM((2, _ACHUNK, _D), jnp.float32),
        pltpu.SemaphoreType.DMA((2,)),
        pltpu.SemaphoreType.DMA((2,)),
        pltpu.VMEM_SHARED((_NPA, _D), jnp.float32),
    ],
)
def _agg_sc(z_hbm, srcs_hbm, dsts_hbm, zeros_hbm, out_hbm,
            sidx, didx, rows_v, gsems, isems, acc_sh):
    c = lax.axis_index("c")
    s = lax.axis_index("s")
    w = c * _NS + s

    def step_pair(p, j):
        for b in range(2):
            pltpu.make_async_copy(
                z_hbm.at[sidx.at[p, j + b]], rows_v.at[b],
                gsems.at[b]).wait()
            pltpu.sync_copy(rows_v.at[b], acc_sh.at[didx.at[p, j + b]],
                            add=True)
            pltpu.async_copy(
                z_hbm.at[sidx.at[p, j + b + 2]], rows_v.at[b], gsems.at[b])

    pltpu.sync_copy(srcs_hbm.at[w, 0], sidx.at[0])
    pltpu.sync_copy(dsts_hbm.at[w, 0], didx.at[0])
    for b in range(2):
        pltpu.async_copy(z_hbm.at[sidx.at[0, b]], rows_v.at[b], gsems.at[b])
    pltpu.async_copy(srcs_hbm.at[w, 1], sidx.at[1], isems.at[1])
    pltpu.async_copy(dsts_hbm.at[w, 1], didx.at[1], isems.at[1])
    pltpu.sync_copy(zeros_hbm.at[pl.ds(s * _RPTA, _RPTA)],
                    acc_sh.at[pl.ds(s * _RPTA, _RPTA)])
    plsc.subcore_barrier()

    for kb in range(_ANBLK):
        p = kb % 2
        q = 1 - p
        step_pair(p, 0)
        if 1 <= kb < _ANBLK - 1:
            pltpu.async_copy(srcs_hbm.at[w, kb + 1], sidx.at[q], isems.at[q])
            pltpu.async_copy(dsts_hbm.at[w, kb + 1], didx.at[q], isems.at[q])

        @pl.loop(2, _ABS, step=2)
        def _(j):
            step_pair(p, j)

        if kb < _ANBLK - 1:
            pltpu.make_async_copy(
                srcs_hbm.at[w, kb + 1], sidx.at[q], isems.at[q]).wait()
            pltpu.make_async_copy(
                dsts_hbm.at[w, kb + 1], didx.at[q], isems.at[q]).wait()

    p = (_ANBLK - 1) % 2
    for b in range(2):
        pltpu.make_async_copy(
            z_hbm.at[sidx.at[p, _ABS + b]], rows_v.at[b], gsems.at[b]).wait()

    plsc.subcore_barrier()
    pltpu.sync_copy(acc_sh.at[pl.ds(s * _RPTA, _RPTA)],
                    out_hbm.at[pl.ds(c * _NPA + s * _RPTA, _RPTA)])


def _dinv_of(dparts):
    deg = jnp.sum(dparts, axis=0)
    return jnp.where(deg > 0, lax.rsqrt(deg), 0.0)


_DEG_SPEC = pl.BlockSpec((_NW, _BLK, 1), lambda i: (0, i, 0))


def _tc_first(x, W1, deg2):
    def body(x_ref, w_ref, d_ref, o_ref):
        dinv = _dinv_of(d_ref[...])
        h = jnp.dot(x_ref[...], w_ref[...], preferred_element_type=jnp.float32)
        o_ref[...] = h * dinv

    return pl.pallas_call(
        body,
        grid=(_GRID,),
        in_specs=[
            pl.BlockSpec((_BLK, _D), lambda i: (i, 0)),
            pl.BlockSpec((_D, _D), lambda i: (0, 0)),
            _DEG_SPEC,
        ],
        out_specs=pl.BlockSpec((_BLK, _D), lambda i: (i, 0)),
        out_shape=jax.ShapeDtypeStruct((_N, _D), jnp.float32),
    )(x, W1, deg2)


def _tc_mid(acc, deg2, b, g, be, W):
    def body(a0_ref, a1_ref, d_ref, b_ref, g_ref, be_ref, w_ref, o_ref):
        dinv = _dinv_of(d_ref[...])
        agg = a0_ref[0] + a1_ref[0]
        y = agg * dinv + b_ref[...]
        y = jnp.maximum(y, 0.0)
        mu = jnp.mean(y, axis=-1, keepdims=True)
        yc = y - mu
        var = jnp.mean(yc * yc, axis=-1, keepdims=True)
        yn = yc * lax.rsqrt(var + 1e-5) * g_ref[...] + be_ref[...]
        o_ref[...] = jnp.dot(yn, w_ref[...],
                             preferred_element_type=jnp.float32) * dinv

    return pl.pallas_call(
        body,
        grid=(_GRID,),
        in_specs=[
            pl.BlockSpec((1, _BLK, _D), lambda i: (0, i, 0)),
            pl.BlockSpec((1, _BLK, _D), lambda i: (1, i, 0)),
            _DEG_SPEC,
            pl.BlockSpec((1, _D), lambda i: (0, 0)),
            pl.BlockSpec((1, _D), lambda i: (0, 0)),
            pl.BlockSpec((1, _D), lambda i: (0, 0)),
            pl.BlockSpec((_D, _D), lambda i: (0, 0)),
        ],
        out_specs=pl.BlockSpec((_BLK, _D), lambda i: (i, 0)),
        out_shape=jax.ShapeDtypeStruct((_N, _D), jnp.float32),
    )(acc, acc, deg2, b, g, be, W)


def _tc_last(acc, deg2, b3, pW1, pb1, pW2, pb2):
    def body(a0_ref, a1_ref, d_ref, b3_ref, pw1_ref, pb1_ref,
             pw2_ref, pb2_ref, emb_ref, out_ref):
        dinv = _dinv_of(d_ref[...])
        agg = a0_ref[0] + a1_ref[0]
        e = agg * dinv + b3_ref[...]
        emb_ref[...] = e
        h = jnp.maximum(e, 0.0)
        h = jnp.dot(h, pw1_ref[...],
                    preferred_element_type=jnp.float32) + pb1_ref[...]
        o = jnp.dot(h, pw2_ref[...],
                    preferred_element_type=jnp.float32) + pb2_ref[...]
        m = jnp.max(o, axis=-1, keepdims=True)
        lse = jnp.log(jnp.sum(jnp.exp(o - m), axis=-1, keepdims=True))
        out_ref[...] = o - m - lse

    return pl.pallas_call(
        body,
        grid=(_GRID,),
        in_specs=[
            pl.BlockSpec((1, _BLK, _D), lambda i: (0, i, 0)),
            pl.BlockSpec((1, _BLK, _D), lambda i: (1, i, 0)),
            _DEG_SPEC,
            pl.BlockSpec((1, _D), lambda i: (0, 0)),
            pl.BlockSpec((_D, _D), lambda i: (0, 0)),
            pl.BlockSpec((1, _D), lambda i: (0, 0)),
            pl.BlockSpec((_D, _OUT), lambda i: (0, 0)),
            pl.BlockSpec((1, _OUT), lambda i: (0, 0)),
        ],
        out_specs=[
            pl.BlockSpec((_BLK, _D), lambda i: (i, 0)),
            pl.BlockSpec((_BLK, _OUT), lambda i: (i, 0)),
        ],
        out_shape=[
            jax.ShapeDtypeStruct((_N, _D), jnp.float32),
            jax.ShapeDtypeStruct((_N, _OUT), jnp.float32),
        ],
    )(acc, acc, deg2, b3, pW1, pb1, pW2, pb2)


def kernel(x, edge_index, W1, b1, W2, b2, W3, b3, g1, be1, g2, be2,
           pW1, pb1, pW2, pb2):
    loop = jnp.arange(_N, dtype=jnp.int32)
    src = jnp.concatenate([edge_index[0].astype(jnp.int32), loop])
    dst = jnp.concatenate([edge_index[1].astype(jnp.int32), loop])
    npad = _EP - dst.shape[0]
    dstsf = jnp.concatenate(
        [dst, jnp.full((npad,), _N, jnp.int32)]).reshape(_NW, _STEPS * _CHUNK)
    apad = _AEP - src.shape[0]
    src_a = jnp.concatenate([src, jnp.zeros((apad,), jnp.int32)])
    dst_a = jnp.concatenate([dst, jnp.full((apad,), _N, jnp.int32)])
    src_t = jnp.concatenate(
        [src_a.reshape(_NW, _ASTEPS, _ACHUNK),
         jnp.zeros((_NW, 2, _ACHUNK), jnp.int32)], axis=1)
    dst_t = jnp.concatenate(
        [dst_a.reshape(_NW, _ASTEPS, _ACHUNK),
         jnp.full((_NW, 2, _ACHUNK), _N, jnp.int32)], axis=1)
    srcs3 = jnp.stack(
        [src_t[:, kb * _ABS:kb * _ABS + _ABS + 2] for kb in range(_ANBLK)],
        axis=1)
    dsts3 = jnp.stack(
        [dst_t[:, kb * _ABS:kb * _ABS + _ABS + 2] for kb in range(_ANBLK)],
        axis=1)

    zeros_deg = jnp.zeros((_NP,), jnp.float32)
    zeros_acc = jnp.zeros((_NPA, _D), jnp.float32)
    b1r = b1.reshape(1, _D)
    b2r = b2.reshape(1, _D)
    b3r = b3.reshape(1, _D)
    g1r = g1.reshape(1, _D)
    be1r = be1.reshape(1, _D)
    g2r = g2.reshape(1, _D)
    be2r = be2.reshape(1, _D)
    pb1r = pb1.reshape(1, _D)
    pb2r = pb2.reshape(1, _OUT)

    deg2 = _deg_sc(dstsf, zeros_deg).reshape(_NW, _NP, 1)
    z0 = _tc_first(x, W1, deg2)
    a0 = _agg_sc(z0, srcs3, dsts3, zeros_acc).reshape(_NC, _NPA, _D)
    z1 = _tc_mid(a0, deg2, b1r, g1r, be1r, W2)
    a1 = _agg_sc(z1, srcs3, dsts3, zeros_acc).reshape(_NC, _NPA, _D)
    z2 = _tc_mid(a1, deg2, b2r, g2r, be2r, W3)
    a2 = _agg_sc(z2, srcs3, dsts3, zeros_acc).reshape(_NC, _NPA, _D)
    emb, out = _tc_last(a2, deg2, b3r, pW1, pb1r, pW2, pb2r)
    return emb, out


# speedup vs baseline: 1.2276x; 1.2276x over previous
import functools

import jax
import jax.numpy as jnp
from jax import lax
from jax.experimental import pallas as pl
from jax.experimental.pallas import tpu as pltpu
from jax.experimental.pallas import tpu_sc as plsc

_N = 10000
_E = 320000
_D = 128
_OUT = 40

_NC = 2
_NS = 16
_NW = _NC * _NS
_CHUNK = 128
_STEPS = 82
_EP = _NW * _STEPS * _CHUNK
_NP = 10240
_ACHUNK = 128
_ASTEPS = 84
_AEP = _NW * _ASTEPS * _ACHUNK
_NPA = 10240
_RPTA = _NPA // _NS

_BLK = 512
_GRID = _NP // _BLK

_sc_mesh = plsc.VectorSubcoreMesh(
    core_axis_name="c", subcore_axis_name="s", num_cores=_NC, num_subcores=_NS)


@functools.partial(
    pl.kernel,
    out_type=jax.ShapeDtypeStruct((_NW * _NP,), jnp.float32),
    mesh=_sc_mesh,
    scratch_types=[
        pltpu.VMEM((_STEPS * _CHUNK,), jnp.int32),
        pltpu.VMEM((_NP,), jnp.float32),
    ],
    compiler_params=pltpu.CompilerParams(needs_layout_passes=False),
)
def _deg_sc(dsts_hbm, zeros_hbm, out_hbm, dst_v, deg_l):
    c = lax.axis_index("c")
    s = lax.axis_index("s")
    w = c * _NS + s
    pltpu.sync_copy(zeros_hbm, deg_l)
    pltpu.sync_copy(dsts_hbm.at[w], dst_v)
    ones = jnp.ones((16,), jnp.float32)

    @pl.loop(0, _STEPS * _CHUNK // 16)
    def _(j):
        idx = dst_v[pl.ds(j * 16, 16)]
        plsc.addupdate_scatter(deg_l, [idx], ones)

    pltpu.sync_copy(deg_l, out_hbm.at[pl.ds(w * _NP, _NP)])


@functools.partial(
    pl.kernel,
    out_type=jax.ShapeDtypeStruct((_NC * _NPA, _D), jnp.float32),
    mesh=_sc_mesh,
    scratch_types=[
        pltpu.VMEM((_ASTEPS, _ACHUNK), jnp.int32),
        pltpu.VMEM((_ASTEPS, _ACHUNK), jnp.int32),
        pltpu.VMEM((_ACHUNK, _D), jnp.float32),
        pltpu.SemaphoreType.DMA,
        pltpu.VMEM_SHARED((_NPA, _D), jnp.float32),
    ],
)
def _agg_sc(z_hbm, srcs_hbm, dsts_hbm, zeros_hbm, out_hbm,
            sidx, didx, rows_v, sem, acc_sh):
    c = lax.axis_index("c")
    s = lax.axis_index("s")
    w = c * _NS + s
    pltpu.sync_copy(srcs_hbm.at[w], sidx)
    pltpu.sync_copy(dsts_hbm.at[w], didx)
    pltpu.sync_copy(zeros_hbm.at[pl.ds(s * _RPTA, _RPTA)],
                    acc_sh.at[pl.ds(s * _RPTA, _RPTA)])
    plsc.subcore_barrier()

    @pl.loop(0, _ASTEPS)
    def _(j):
        pltpu.async_copy(z_hbm.at[sidx.at[j]], rows_v, sem).wait()
        pltpu.sync_copy(rows_v, acc_sh.at[didx.at[j]], add=True)

    plsc.subcore_barrier()
    pltpu.sync_copy(acc_sh.at[pl.ds(s * _RPTA, _RPTA)],
                    out_hbm.at[pl.ds(c * _NPA + s * _RPTA, _RPTA)])


def _dinv_of(dparts):
    deg = jnp.sum(dparts, axis=0)
    return jnp.where(deg > 0, lax.rsqrt(deg), 0.0)


_DEG_SPEC = pl.BlockSpec((_NW, _BLK, 1), lambda i: (0, i, 0))


def _tc_first(x, W1, deg2):
    def body(x_ref, w_ref, d_ref, o_ref):
        dinv = _dinv_of(d_ref[...])
        h = jnp.dot(x_ref[...], w_ref[...], preferred_element_type=jnp.float32)
        o_ref[...] = h * dinv

    return pl.pallas_call(
        body,
        grid=(_GRID,),
        in_specs=[
            pl.BlockSpec((_BLK, _D), lambda i: (i, 0)),
            pl.BlockSpec((_D, _D), lambda i: (0, 0)),
            _DEG_SPEC,
        ],
        out_specs=pl.BlockSpec((_BLK, _D), lambda i: (i, 0)),
        out_shape=jax.ShapeDtypeStruct((_N, _D), jnp.float32),
    )(x, W1, deg2)


def _tc_mid(acc, deg2, b, g, be, W):
    def body(a0_ref, a1_ref, d_ref, b_ref, g_ref, be_ref, w_ref, o_ref):
        dinv = _dinv_of(d_ref[...])
        agg = a0_ref[0] + a1_ref[0]
        y = agg * dinv + b_ref[...]
        y = jnp.maximum(y, 0.0)
        mu = jnp.mean(y, axis=-1, keepdims=True)
        yc = y - mu
        var = jnp.mean(yc * yc, axis=-1, keepdims=True)
        yn = yc * lax.rsqrt(var + 1e-5) * g_ref[...] + be_ref[...]
        o_ref[...] = jnp.dot(yn, w_ref[...],
                             preferred_element_type=jnp.float32) * dinv

    return pl.pallas_call(
        body,
        grid=(_GRID,),
        in_specs=[
            pl.BlockSpec((1, _BLK, _D), lambda i: (0, i, 0)),
            pl.BlockSpec((1, _BLK, _D), lambda i: (1, i, 0)),
            _DEG_SPEC,
            pl.BlockSpec((1, _D), lambda i: (0, 0)),
            pl.BlockSpec((1, _D), lambda i: (0, 0)),
            pl.BlockSpec((1, _D), lambda i: (0, 0)),
            pl.BlockSpec((_D, _D), lambda i: (0, 0)),
        ],
        out_specs=pl.BlockSpec((_BLK, _D), lambda i: (i, 0)),
        out_shape=jax.ShapeDtypeStruct((_N, _D), jnp.float32),
    )(acc, acc, deg2, b, g, be, W)


def _tc_last(acc, deg2, b3, pW1, pb1, pW2, pb2):
    def body(a0_ref, a1_ref, d_ref, b3_ref, pw1_ref, pb1_ref,
             pw2_ref, pb2_ref, emb_ref, out_ref):
        dinv = _dinv_of(d_ref[...])
        agg = a0_ref[0] + a1_ref[0]
        e = agg * dinv + b3_ref[...]
        emb_ref[...] = e
        h = jnp.maximum(e, 0.0)
        h = jnp.dot(h, pw1_ref[...],
                    preferred_element_type=jnp.float32) + pb1_ref[...]
        o = jnp.dot(h, pw2_ref[...],
                    preferred_element_type=jnp.float32) + pb2_ref[...]
        m = jnp.max(o, axis=-1, keepdims=True)
        lse = jnp.log(jnp.sum(jnp.exp(o - m), axis=-1, keepdims=True))
        out_ref[...] = o - m - lse

    return pl.pallas_call(
        body,
        grid=(_GRID,),
        in_specs=[
            pl.BlockSpec((1, _BLK, _D), lambda i: (0, i, 0)),
            pl.BlockSpec((1, _BLK, _D), lambda i: (1, i, 0)),
            _DEG_SPEC,
            pl.BlockSpec((1, _D), lambda i: (0, 0)),
            pl.BlockSpec((_D, _D), lambda i: (0, 0)),
            pl.BlockSpec((1, _D), lambda i: (0, 0)),
            pl.BlockSpec((_D, _OUT), lambda i: (0, 0)),
            pl.BlockSpec((1, _OUT), lambda i: (0, 0)),
        ],
        out_specs=[
            pl.BlockSpec((_BLK, _D), lambda i: (i, 0)),
            pl.BlockSpec((_BLK, _OUT), lambda i: (i, 0)),
        ],
        out_shape=[
            jax.ShapeDtypeStruct((_N, _D), jnp.float32),
            jax.ShapeDtypeStruct((_N, _OUT), jnp.float32),
        ],
    )(acc, acc, deg2, b3, pW1, pb1, pW2, pb2)


def kernel(x, edge_index, W1, b1, W2, b2, W3, b3, g1, be1, g2, be2,
           pW1, pb1, pW2, pb2):
    loop = jnp.arange(_N, dtype=jnp.int32)
    src = jnp.concatenate([edge_index[0].astype(jnp.int32), loop])
    dst = jnp.concatenate([edge_index[1].astype(jnp.int32), loop])
    npad = _EP - dst.shape[0]
    dstsf = jnp.concatenate(
        [dst, jnp.full((npad,), _N, jnp.int32)]).reshape(_NW, _STEPS * _CHUNK)
    apad = _AEP - src.shape[0]
    src_a = jnp.concatenate([src, jnp.zeros((apad,), jnp.int32)])
    dst_a = jnp.concatenate([dst, jnp.full((apad,), _N, jnp.int32)])
    srcs3 = src_a.reshape(_NW, _ASTEPS, _ACHUNK)
    dsts3 = dst_a.reshape(_NW, _ASTEPS, _ACHUNK)

    zeros_deg = jnp.zeros((_NP,), jnp.float32)
    zeros_acc = jnp.zeros((_NPA, _D), jnp.float32)
    b1r = b1.reshape(1, _D)
    b2r = b2.reshape(1, _D)
    b3r = b3.reshape(1, _D)
    g1r = g1.reshape(1, _D)
    be1r = be1.reshape(1, _D)
    g2r = g2.reshape(1, _D)
    be2r = be2.reshape(1, _D)
    pb1r = pb1.reshape(1, _D)
    pb2r = pb2.reshape(1, _OUT)

    deg2 = _deg_sc(dstsf, zeros_deg).reshape(_NW, _NP, 1)
    z0 = _tc_first(x, W1, deg2)
    a0 = _agg_sc(z0, srcs3, dsts3, zeros_acc).reshape(_NC, _NPA, _D)
    z1 = _tc_mid(a0, deg2, b1r, g1r, be1r, W2)
    a1 = _agg_sc(z1, srcs3, dsts3, zeros_acc).reshape(_NC, _NPA, _D)
    z2 = _tc_mid(a1, deg2, b2r, g2r, be2r, W3)
    a2 = _agg_sc(z2, srcs3, dsts3, zeros_acc).reshape(_NC, _NPA, _D)
    emb, out = _tc_last(a2, deg2, b3r, pW1, pb1r, pW2, pb2r)
    return emb, out


# speedup vs baseline: 1.3526x; 1.1018x over previous
import functools

import jax
import jax.numpy as jnp
from jax import lax
from jax.experimental import pallas as pl
from jax.experimental.pallas import tpu as pltpu
from jax.experimental.pallas import tpu_sc as plsc

_N = 10000
_E = 320000
_D = 128
_OUT = 40

_NC = 2
_NS = 16
_NW = _NC * _NS
_CHUNK = 128
_STEPS = 82
_EP = _NW * _STEPS * _CHUNK
_NP = 10240
_ACHUNK = 128
_ASTEPS = 84
_AEP = _NW * _ASTEPS * _ACHUNK
_NPA = 10240
_RPTA = _NPA // _NS

_BLK = 512
_GRID = _NP // _BLK

_sc_mesh = plsc.VectorSubcoreMesh(
    core_axis_name="c", subcore_axis_name="s", num_cores=_NC, num_subcores=_NS)


@functools.partial(
    pl.kernel,
    out_type=jax.ShapeDtypeStruct((_NW * _NP,), jnp.float32),
    mesh=_sc_mesh,
    scratch_types=[
        pltpu.VMEM((_STEPS * _CHUNK,), jnp.int32),
        pltpu.VMEM((_NP,), jnp.float32),
    ],
    compiler_params=pltpu.CompilerParams(needs_layout_passes=False),
)
def _deg_sc(dsts_hbm, zeros_hbm, out_hbm, dst_v, deg_l):
    c = lax.axis_index("c")
    s = lax.axis_index("s")
    w = c * _NS + s
    pltpu.sync_copy(zeros_hbm, deg_l)
    pltpu.sync_copy(dsts_hbm.at[w], dst_v)
    ones = jnp.ones((16,), jnp.float32)

    @pl.loop(0, _STEPS * _CHUNK // 16)
    def _(j):
        idx = dst_v[pl.ds(j * 16, 16)]
        plsc.addupdate_scatter(deg_l, [idx], ones)

    pltpu.sync_copy(deg_l, out_hbm.at[pl.ds(w * _NP, _NP)])


@functools.partial(
    pl.kernel,
    out_type=jax.ShapeDtypeStruct((_NC * _NPA, _D), jnp.float32),
    mesh=_sc_mesh,
    scratch_types=[
        pltpu.VMEM((_ASTEPS, _ACHUNK), jnp.int32),
        pltpu.VMEM((_ASTEPS, _ACHUNK), jnp.int32),
        pltpu.VMEM((_ACHUNK, _D), jnp.float32),
        pltpu.SemaphoreType.DMA,
        pltpu.VMEM_SHARED((_NPA, _D), jnp.float32),
    ],
)
def _agg_sc(z_hbm, srcs_hbm, dsts_hbm, zeros_hbm, out_hbm,
            sidx, didx, rows_v, sem, acc_sh):
    c = lax.axis_index("c")
    s = lax.axis_index("s")
    w = c * _NS + s
    pltpu.sync_copy(srcs_hbm.at[w], sidx)
    pltpu.sync_copy(dsts_hbm.at[w], didx)
    pltpu.sync_copy(zeros_hbm.at[pl.ds(s * _RPTA, _RPTA)],
                    acc_sh.at[pl.ds(s * _RPTA, _RPTA)])
    plsc.subcore_barrier()

    @pl.loop(0, _ASTEPS)
    def _(j):
        pltpu.async_copy(z_hbm.at[sidx.at[j]], rows_v, sem).wait()
        pltpu.sync_copy(rows_v, acc_sh.at[didx.at[j]], add=True)

    plsc.subcore_barrier()
    pltpu.sync_copy(acc_sh.at[pl.ds(s * _RPTA, _RPTA)],
                    out_hbm.at[pl.ds(c * _NPA + s * _RPTA, _RPTA)])


def _dinv_of(dparts):
    deg = jnp.sum(dparts, axis=0)
    return jnp.where(deg > 0, lax.rsqrt(deg), 0.0)


_DEG_SPEC = pl.BlockSpec((_NW, _BLK, 1), lambda i: (0, i, 0))


def _tc_first(x, W1, deg2):
    def body(x_ref, w_ref, d_ref, o_ref):
        dinv = _dinv_of(d_ref[...])
        h = jnp.dot(x_ref[...], w_ref[...], preferred_element_type=jnp.float32)
        o_ref[...] = h * dinv

    return pl.pallas_call(
        body,
        grid=(_GRID,),
        in_specs=[
            pl.BlockSpec((_BLK, _D), lambda i: (i, 0)),
            pl.BlockSpec((_D, _D), lambda i: (0, 0)),
            _DEG_SPEC,
        ],
        out_specs=pl.BlockSpec((_BLK, _D), lambda i: (i, 0)),
        out_shape=jax.ShapeDtypeStruct((_N, _D), jnp.float32),
    )(x, W1, deg2)


def _tc_mid(acc, deg2, b, g, be, W):
    def body(a0_ref, a1_ref, d_ref, b_ref, g_ref, be_ref, w_ref, o_ref):
        dinv = _dinv_of(d_ref[...])
        agg = a0_ref[0] + a1_ref[0]
        y = agg * dinv + b_ref[...]
        y = jnp.maximum(y, 0.0)
        mu = jnp.mean(y, axis=-1, keepdims=True)
        yc = y - mu
        var = jnp.mean(yc * yc, axis=-1, keepdims=True)
        yn = yc * lax.rsqrt(var + 1e-5) * g_ref[...] + be_ref[...]
        o_ref[...] = jnp.dot(yn, w_ref[...],
                             preferred_element_type=jnp.float32) * dinv

    return pl.pallas_call(
        body,
        grid=(_GRID,),
        in_specs=[
            pl.BlockSpec((1, _BLK, _D), lambda i: (0, i, 0)),
            pl.BlockSpec((1, _BLK, _D), lambda i: (1, i, 0)),
            _DEG_SPEC,
            pl.BlockSpec((1, _D), lambda i: (0, 0)),
            pl.BlockSpec((1, _D), lambda i: (0, 0)),
            pl.BlockSpec((1, _D), lambda i: (0, 0)),
            pl.BlockSpec((_D, _D), lambda i: (0, 0)),
        ],
        out_specs=pl.BlockSpec((_BLK, _D), lambda i: (i, 0)),
        out_shape=jax.ShapeDtypeStruct((_N, _D), jnp.float32),
    )(acc, acc, deg2, b, g, be, W)


def _tc_last(acc, deg2, b3, pW1, pb1, pW2, pb2):
    def body(a0_ref, a1_ref, d_ref, b3_ref, pw1_ref, pb1_ref,
             pw2_ref, pb2_ref, emb_ref, out_ref):
        dinv = _dinv_of(d_ref[...])
        agg = a0_ref[0] + a1_ref[0]
        e = agg * dinv + b3_ref[...]
        emb_ref[...] = e
        h = jnp.maximum(e, 0.0)
        h = jnp.dot(h, pw1_ref[...],
                    preferred_element_type=jnp.float32) + pb1_ref[...]
        o = jnp.dot(h, pw2_ref[...],
                    preferred_element_type=jnp.float32) + pb2_ref[...]
        m = jnp.max(o, axis=-1, keepdims=True)
        lse = jnp.log(jnp.sum(jnp.exp(o - m), axis=-1, keepdims=True))
        out_ref[...] = o - m - lse

    return pl.pallas_call(
        body,
        grid=(_GRID,),
        in_specs=[
            pl.BlockSpec((1, _BLK, _D), lambda i: (0, i, 0)),
            pl.BlockSpec((1, _BLK, _D), lambda i: (1, i, 0)),
            _DEG_SPEC,
            pl.BlockSpec((1, _D), lambda i: (0, 0)),
            pl.BlockSpec((_D, _D), lambda i: (0, 0)),
            pl.BlockSpec((1, _D), lambda i: (0, 0)),
            pl.BlockSpec((_D, _OUT), lambda i: (0, 0)),
            pl.BlockSpec((1, _OUT), lambda i: (0, 0)),
        ],
        out_specs=[
            pl.BlockSpec((_BLK, _D), lambda i: (i, 0)),
            pl.BlockSpec((_BLK, _OUT), lambda i: (i, 0)),
        ],
        out_shape=[
            jax.ShapeDtypeStruct((_N, _D), jnp.float32),
            jax.ShapeDtypeStruct((_N, _OUT), jnp.float32),
        ],
    )(acc, acc, deg2, b3, pW1, pb1, pW2, pb2)


def kernel(x, edge_index, W1, b1, W2, b2, W3, b3, g1, be1, g2, be2,
           pW1, pb1, pW2, pb2):
    loop = jnp.arange(_N, dtype=jnp.int32)
    src = jnp.concatenate([edge_index[0].astype(jnp.int32), loop])
    dst = jnp.concatenate([edge_index[1].astype(jnp.int32), loop])
    npad = _EP - dst.shape[0]
    dstsf = jnp.concatenate(
        [dst, jnp.full((npad,), _N, jnp.int32)]).reshape(_NW, _STEPS * _CHUNK)
    apad = _AEP - src.shape[0]
    src_a = jnp.concatenate([src, jnp.zeros((apad,), jnp.int32)])
    dst_a = jnp.concatenate([dst, jnp.full((apad,), _N, jnp.int32)])
    srcs3 = src_a.reshape(-1, _NW).T.reshape(_NW, _ASTEPS, _ACHUNK)
    dsts3 = dst_a.reshape(-1, _NW).T.reshape(_NW, _ASTEPS, _ACHUNK)

    zeros_deg = jnp.zeros((_NP,), jnp.float32)
    zeros_acc = jnp.zeros((_NPA, _D), jnp.float32)
    b1r = b1.reshape(1, _D)
    b2r = b2.reshape(1, _D)
    b3r = b3.reshape(1, _D)
    g1r = g1.reshape(1, _D)
    be1r = be1.reshape(1, _D)
    g2r = g2.reshape(1, _D)
    be2r = be2.reshape(1, _D)
    pb1r = pb1.reshape(1, _D)
    pb2r = pb2.reshape(1, _OUT)

    deg2 = _deg_sc(dstsf, zeros_deg).reshape(_NW, _NP, 1)
    z0 = _tc_first(x, W1, deg2)
    a0 = _agg_sc(z0, srcs3, dsts3, zeros_acc).reshape(_NC, _NPA, _D)
    z1 = _tc_mid(a0, deg2, b1r, g1r, be1r, W2)
    a1 = _agg_sc(z1, srcs3, dsts3, zeros_acc).reshape(_NC, _NPA, _D)
    z2 = _tc_mid(a1, deg2, b2r, g2r, be2r, W3)
    a2 = _agg_sc(z2, srcs3, dsts3, zeros_acc).reshape(_NC, _NPA, _D)
    emb, out = _tc_last(a2, deg2, b3r, pW1, pb1r, pW2, pb2r)
    return emb, out


# speedup vs baseline: 2.2623x; 1.6726x over previous
import functools

import jax
import jax.numpy as jnp
from jax import lax
from jax.experimental import pallas as pl
from jax.experimental.pallas import tpu as pltpu
from jax.experimental.pallas import tpu_sc as plsc

_N = 10000
_E = 320000
_D = 128
_OUT = 40

_NC = 2
_NS = 16
_NW = _NC * _NS
_CHUNK = 128
_STEPS = 82
_EP = _NW * _STEPS * _CHUNK
_NP = 10240
_ACHUNK = 128
_ASTEPS = 82
_AEP = _NW * _ASTEPS * _ACHUNK
_NPA = 10240
_RPTA = _NPA // _NS

_BLK = 512
_GRID = _NP // _BLK

_sc_mesh = plsc.VectorSubcoreMesh(
    core_axis_name="c", subcore_axis_name="s", num_cores=_NC, num_subcores=_NS)


@functools.partial(
    pl.kernel,
    out_type=jax.ShapeDtypeStruct((_NW * _NP,), jnp.float32),
    mesh=_sc_mesh,
    scratch_types=[
        pltpu.VMEM((_STEPS * _CHUNK,), jnp.int32),
        pltpu.VMEM((_NP,), jnp.float32),
    ],
    compiler_params=pltpu.CompilerParams(needs_layout_passes=False),
)
def _deg_sc(dsts_hbm, zeros_hbm, out_hbm, dst_v, deg_l):
    c = lax.axis_index("c")
    s = lax.axis_index("s")
    w = c * _NS + s
    pltpu.sync_copy(zeros_hbm, deg_l)
    pltpu.sync_copy(dsts_hbm.at[w], dst_v)
    ones = jnp.ones((16,), jnp.float32)

    @pl.loop(0, _STEPS * _CHUNK // 16)
    def _(j):
        idx = dst_v[pl.ds(j * 16, 16)]
        plsc.addupdate_scatter(deg_l, [idx], ones)

    pltpu.sync_copy(deg_l, out_hbm.at[pl.ds(w * _NP, _NP)])


@functools.partial(
    pl.kernel,
    out_type=jax.ShapeDtypeStruct((_NC * _NPA, _D), jnp.float32),
    mesh=_sc_mesh,
    scratch_types=[
        pltpu.VMEM((_ASTEPS, _ACHUNK), jnp.int32),
        pltpu.VMEM((_ASTEPS, _ACHUNK), jnp.int32),
        pltpu.VMEM((_ACHUNK, _D), jnp.float32),
        pltpu.SemaphoreType.DMA,
        pltpu.VMEM_SHARED((_NPA, _D), jnp.float32),
    ],
)
def _agg_sc(z_hbm, srcs_hbm, dsts_hbm, zeros_hbm, out_hbm,
            sidx, didx, rows_v, sem, acc_sh):
    c = lax.axis_index("c")
    s = lax.axis_index("s")
    w = c * _NS + s
    pltpu.sync_copy(srcs_hbm.at[w], sidx)
    pltpu.sync_copy(dsts_hbm.at[w], didx)
    pltpu.sync_copy(zeros_hbm.at[pl.ds(s * _RPTA, _RPTA)],
                    acc_sh.at[pl.ds(s * _RPTA, _RPTA)])
    plsc.subcore_barrier()

    @pl.loop(0, _ASTEPS)
    def _(j):
        pltpu.async_copy(z_hbm.at[sidx.at[j]], rows_v, sem).wait()
        pltpu.sync_copy(rows_v, acc_sh.at[didx.at[j]], add=True)

    plsc.subcore_barrier()
    pltpu.sync_copy(acc_sh.at[pl.ds(s * _RPTA, _RPTA)],
                    out_hbm.at[pl.ds(c * _NPA + s * _RPTA, _RPTA)])


_DINV_SPEC = pl.BlockSpec((_BLK, _D), lambda i: (i, 0))


def _tc_dinv(deg2):
    def body(d_ref, o_ref):
        deg = jnp.sum(d_ref[...], axis=0)
        dinv = jnp.where(deg > 0, lax.rsqrt(deg), 0.0)
        o_ref[...] = jnp.broadcast_to(dinv, (_BLK, _D))

    return pl.pallas_call(
        body,
        grid=(_GRID,),
        in_specs=[pl.BlockSpec((_NW, _BLK, 1), lambda i: (0, i, 0))],
        out_specs=pl.BlockSpec((_BLK, _D), lambda i: (i, 0)),
        out_shape=jax.ShapeDtypeStruct((_NP, _D), jnp.float32),
    )(deg2)


def _tc_first(x, W1, dinv_b):
    def body(x_ref, w_ref, d_ref, o_ref):
        h = jnp.dot(x_ref[...], w_ref[...], preferred_element_type=jnp.float32)
        o_ref[...] = h * d_ref[...]

    return pl.pallas_call(
        body,
        grid=(_GRID,),
        in_specs=[
            pl.BlockSpec((_BLK, _D), lambda i: (i, 0)),
            pl.BlockSpec((_D, _D), lambda i: (0, 0)),
            _DINV_SPEC,
        ],
        out_specs=pl.BlockSpec((_BLK, _D), lambda i: (i, 0)),
        out_shape=jax.ShapeDtypeStruct((_N, _D), jnp.float32),
    )(x, W1, dinv_b)


def _tc_mid(acc, dinv_b, b, g, be, W):
    def body(a0_ref, a1_ref, d_ref, b_ref, g_ref, be_ref, w_ref, o_ref):
        dinv = d_ref[...]
        agg = a0_ref[0] + a1_ref[0]
        y = agg * dinv + b_ref[...]
        y = jnp.maximum(y, 0.0)
        mu = jnp.mean(y, axis=-1, keepdims=True)
        yc = y - mu
        var = jnp.mean(yc * yc, axis=-1, keepdims=True)
        yn = yc * lax.rsqrt(var + 1e-5) * g_ref[...] + be_ref[...]
        o_ref[...] = jnp.dot(yn, w_ref[...],
                             preferred_element_type=jnp.float32) * dinv

    return pl.pallas_call(
        body,
        grid=(_GRID,),
        in_specs=[
            pl.BlockSpec((1, _BLK, _D), lambda i: (0, i, 0)),
            pl.BlockSpec((1, _BLK, _D), lambda i: (1, i, 0)),
            _DINV_SPEC,
            pl.BlockSpec((1, _D), lambda i: (0, 0)),
            pl.BlockSpec((1, _D), lambda i: (0, 0)),
            pl.BlockSpec((1, _D), lambda i: (0, 0)),
            pl.BlockSpec((_D, _D), lambda i: (0, 0)),
        ],
        out_specs=pl.BlockSpec((_BLK, _D), lambda i: (i, 0)),
        out_shape=jax.ShapeDtypeStruct((_N, _D), jnp.float32),
    )(acc, acc, dinv_b, b, g, be, W)


def _tc_last(acc, dinv_b, b3, pW1, pb1, pW2, pb2):
    def body(a0_ref, a1_ref, d_ref, b3_ref, pw1_ref, pb1_ref,
             pw2_ref, pb2_ref, emb_ref, out_ref):
        dinv = d_ref[...]
        agg = a0_ref[0] + a1_ref[0]
        e = agg * dinv + b3_ref[...]
        emb_ref[...] = e
        h = jnp.maximum(e, 0.0)
        h = jnp.dot(h, pw1_ref[...],
                    preferred_element_type=jnp.float32) + pb1_ref[...]
        o = jnp.dot(h, pw2_ref[...],
                    preferred_element_type=jnp.float32) + pb2_ref[...]
        m = jnp.max(o, axis=-1, keepdims=True)
        lse = jnp.log(jnp.sum(jnp.exp(o - m), axis=-1, keepdims=True))
        out_ref[...] = o - m - lse

    return pl.pallas_call(
        body,
        grid=(_GRID,),
        in_specs=[
            pl.BlockSpec((1, _BLK, _D), lambda i: (0, i, 0)),
            pl.BlockSpec((1, _BLK, _D), lambda i: (1, i, 0)),
            _DINV_SPEC,
            pl.BlockSpec((1, _D), lambda i: (0, 0)),
            pl.BlockSpec((_D, _D), lambda i: (0, 0)),
            pl.BlockSpec((1, _D), lambda i: (0, 0)),
            pl.BlockSpec((_D, _OUT), lambda i: (0, 0)),
            pl.BlockSpec((1, _OUT), lambda i: (0, 0)),
        ],
        out_specs=[
            pl.BlockSpec((_BLK, _D), lambda i: (i, 0)),
            pl.BlockSpec((_BLK, _OUT), lambda i: (i, 0)),
        ],
        out_shape=[
            jax.ShapeDtypeStruct((_N, _D), jnp.float32),
            jax.ShapeDtypeStruct((_N, _OUT), jnp.float32),
        ],
    )(acc, acc, dinv_b, b3, pW1, pb1, pW2, pb2)


def kernel(x, edge_index, W1, b1, W2, b2, W3, b3, g1, be1, g2, be2,
           pW1, pb1, pW2, pb2):
    loop = jnp.arange(_N, dtype=jnp.int32)
    src = jnp.concatenate([edge_index[0].astype(jnp.int32), loop])
    dst = jnp.concatenate([edge_index[1].astype(jnp.int32), loop])
    npad = _EP - dst.shape[0]
    dstsf = jnp.concatenate(
        [dst, jnp.full((npad,), _N, jnp.int32)]).reshape(_NW, _STEPS * _CHUNK)
    apad = _AEP - src.shape[0]
    src_a = jnp.concatenate([src, jnp.zeros((apad,), jnp.int32)])
    dst_a = jnp.concatenate([dst, jnp.full((apad,), _N, jnp.int32)])
    srcs3 = src_a.reshape(-1, _NW).T.reshape(_NW, _ASTEPS, _ACHUNK)
    dsts3 = dst_a.reshape(-1, _NW).T.reshape(_NW, _ASTEPS, _ACHUNK)

    zeros_deg = jnp.zeros((_NP,), jnp.float32)
    zeros_acc = jnp.zeros((_NPA, _D), jnp.float32)
    b1r = b1.reshape(1, _D)
    b2r = b2.reshape(1, _D)
    b3r = b3.reshape(1, _D)
    g1r = g1.reshape(1, _D)
    be1r = be1.reshape(1, _D)
    g2r = g2.reshape(1, _D)
    be2r = be2.reshape(1, _D)
    pb1r = pb1.reshape(1, _D)
    pb2r = pb2.reshape(1, _OUT)

    deg2 = _deg_sc(dstsf, zeros_deg).reshape(_NW, _NP, 1)
    dinv_b = _tc_dinv(deg2)
    z0 = _tc_first(x, W1, dinv_b)
    a0 = _agg_sc(z0, srcs3, dsts3, zeros_acc).reshape(_NC, _NPA, _D)
    z1 = _tc_mid(a0, dinv_b, b1r, g1r, be1r, W2)
    a1 = _agg_sc(z1, srcs3, dsts3, zeros_acc).reshape(_NC, _NPA, _D)
    z2 = _tc_mid(a1, dinv_b, b2r, g2r, be2r, W3)
    a2 = _agg_sc(z2, srcs3, dsts3, zeros_acc).reshape(_NC, _NPA, _D)
    emb, out = _tc_last(a2, dinv_b, b3r, pW1, pb1r, pW2, pb2r)
    return emb, out


# speedup vs baseline: 2.2958x; 1.0148x over previous
import functools

import jax
import jax.numpy as jnp
from jax import lax
from jax.experimental import pallas as pl
from jax.experimental.pallas import tpu as pltpu
from jax.experimental.pallas import tpu_sc as plsc

_N = 10000
_E = 320000
_D = 128
_OUT = 40

_NC = 2
_NS = 16
_NW = _NC * _NS
_CHUNK = 128
_STEPS = 82
_EP = _NW * _STEPS * _CHUNK
_NP = 10240
_ACHUNK = 128
_ASTEPS = 82
_AEP = _NW * _ASTEPS * _ACHUNK
_NPA = 10240
_RPTA = _NPA // _NS

_BLK = 1024
_GRID = _NP // _BLK

_sc_mesh = plsc.VectorSubcoreMesh(
    core_axis_name="c", subcore_axis_name="s", num_cores=_NC, num_subcores=_NS)


@functools.partial(
    pl.kernel,
    out_type=jax.ShapeDtypeStruct((_NW * _NP,), jnp.float32),
    mesh=_sc_mesh,
    scratch_types=[
        pltpu.VMEM((_STEPS * _CHUNK,), jnp.int32),
        pltpu.VMEM((_NP,), jnp.float32),
    ],
    compiler_params=pltpu.CompilerParams(needs_layout_passes=False),
)
def _deg_sc(dsts_hbm, zeros_hbm, out_hbm, dst_v, deg_l):
    c = lax.axis_index("c")
    s = lax.axis_index("s")
    w = c * _NS + s
    pltpu.sync_copy(zeros_hbm, deg_l)
    pltpu.sync_copy(dsts_hbm.at[w], dst_v)
    ones = jnp.ones((16,), jnp.float32)

    @pl.loop(0, _STEPS * _CHUNK // 16)
    def _(j):
        idx = dst_v[pl.ds(j * 16, 16)]
        plsc.addupdate_scatter(deg_l, [idx], ones)

    pltpu.sync_copy(deg_l, out_hbm.at[pl.ds(w * _NP, _NP)])


@functools.partial(
    pl.kernel,
    out_type=jax.ShapeDtypeStruct((_NC * _NPA, _D), jnp.float32),
    mesh=_sc_mesh,
    scratch_types=[
        pltpu.VMEM((_ASTEPS, _ACHUNK), jnp.int32),
        pltpu.VMEM((_ASTEPS, _ACHUNK), jnp.int32),
        pltpu.VMEM((_ACHUNK, _D), jnp.float32),
        pltpu.SemaphoreType.DMA,
        pltpu.VMEM_SHARED((_NPA, _D), jnp.float32),
    ],
)
def _agg_sc(z_hbm, srcs_hbm, dsts_hbm, zeros_hbm, out_hbm,
            sidx, didx, rows_v, sem, acc_sh):
    c = lax.axis_index("c")
    s = lax.axis_index("s")
    w = c * _NS + s
    pltpu.sync_copy(srcs_hbm.at[w], sidx)
    pltpu.sync_copy(dsts_hbm.at[w], didx)
    pltpu.sync_copy(zeros_hbm.at[pl.ds(s * _RPTA, _RPTA)],
                    acc_sh.at[pl.ds(s * _RPTA, _RPTA)])
    plsc.subcore_barrier()

    @pl.loop(0, _ASTEPS)
    def _(j):
        pltpu.async_copy(z_hbm.at[sidx.at[j]], rows_v, sem).wait()
        pltpu.sync_copy(rows_v, acc_sh.at[didx.at[j]], add=True)

    plsc.subcore_barrier()
    pltpu.sync_copy(acc_sh.at[pl.ds(s * _RPTA, _RPTA)],
                    out_hbm.at[pl.ds(c * _NPA + s * _RPTA, _RPTA)])


_DINV_SPEC = pl.BlockSpec((_BLK, _D), lambda i: (i, 0))


def _tc_dinv(deg2):
    def body(d_ref, o_ref):
        deg = jnp.sum(d_ref[...], axis=0)
        dinv = jnp.where(deg > 0, lax.rsqrt(deg), 0.0)
        o_ref[...] = jnp.broadcast_to(dinv, (_BLK, _D))

    return pl.pallas_call(
        body,
        grid=(_GRID,),
        in_specs=[pl.BlockSpec((_NW, _BLK, 1), lambda i: (0, i, 0))],
        out_specs=pl.BlockSpec((_BLK, _D), lambda i: (i, 0)),
        out_shape=jax.ShapeDtypeStruct((_NP, _D), jnp.float32),
    )(deg2)


def _tc_first(x, W1, dinv_b):
    def body(x_ref, w_ref, d_ref, o_ref):
        h = jnp.dot(x_ref[...], w_ref[...], preferred_element_type=jnp.float32)
        o_ref[...] = h * d_ref[...]

    return pl.pallas_call(
        body,
        grid=(_GRID,),
        in_specs=[
            pl.BlockSpec((_BLK, _D), lambda i: (i, 0)),
            pl.BlockSpec((_D, _D), lambda i: (0, 0)),
            _DINV_SPEC,
        ],
        out_specs=pl.BlockSpec((_BLK, _D), lambda i: (i, 0)),
        out_shape=jax.ShapeDtypeStruct((_N, _D), jnp.float32),
    )(x, W1, dinv_b)


def _tc_mid(acc, dinv_b, b, g, be, W):
    def body(a0_ref, a1_ref, d_ref, b_ref, g_ref, be_ref, w_ref, o_ref):
        dinv = d_ref[...]
        agg = a0_ref[0] + a1_ref[0]
        y = agg * dinv + b_ref[...]
        y = jnp.maximum(y, 0.0)
        mu = jnp.mean(y, axis=-1, keepdims=True)
        yc = y - mu
        var = jnp.mean(yc * yc, axis=-1, keepdims=True)
        yn = yc * lax.rsqrt(var + 1e-5) * g_ref[...] + be_ref[...]
        o_ref[...] = jnp.dot(yn, w_ref[...],
                             preferred_element_type=jnp.float32) * dinv

    return pl.pallas_call(
        body,
        grid=(_GRID,),
        in_specs=[
            pl.BlockSpec((1, _BLK, _D), lambda i: (0, i, 0)),
            pl.BlockSpec((1, _BLK, _D), lambda i: (1, i, 0)),
            _DINV_SPEC,
            pl.BlockSpec((1, _D), lambda i: (0, 0)),
            pl.BlockSpec((1, _D), lambda i: (0, 0)),
            pl.BlockSpec((1, _D), lambda i: (0, 0)),
            pl.BlockSpec((_D, _D), lambda i: (0, 0)),
        ],
        out_specs=pl.BlockSpec((_BLK, _D), lambda i: (i, 0)),
        out_shape=jax.ShapeDtypeStruct((_N, _D), jnp.float32),
    )(acc, acc, dinv_b, b, g, be, W)


def _tc_last(acc, dinv_b, b3, pW1, pb1, pW2, pb2):
    def body(a0_ref, a1_ref, d_ref, b3_ref, pw1_ref, pb1_ref,
             pw2_ref, pb2_ref, emb_ref, out_ref):
        dinv = d_ref[...]
        agg = a0_ref[0] + a1_ref[0]
        e = agg * dinv + b3_ref[...]
        emb_ref[...] = e
        h = jnp.maximum(e, 0.0)
        h = jnp.dot(h, pw1_ref[...],
                    preferred_element_type=jnp.float32) + pb1_ref[...]
        o = jnp.dot(h, pw2_ref[...],
                    preferred_element_type=jnp.float32) + pb2_ref[...]
        m = jnp.max(o, axis=-1, keepdims=True)
        lse = jnp.log(jnp.sum(jnp.exp(o - m), axis=-1, keepdims=True))
        out_ref[...] = o - m - lse

    return pl.pallas_call(
        body,
        grid=(_GRID,),
        in_specs=[
            pl.BlockSpec((1, _BLK, _D), lambda i: (0, i, 0)),
            pl.BlockSpec((1, _BLK, _D), lambda i: (1, i, 0)),
            _DINV_SPEC,
            pl.BlockSpec((1, _D), lambda i: (0, 0)),
            pl.BlockSpec((_D, _D), lambda i: (0, 0)),
            pl.BlockSpec((1, _D), lambda i: (0, 0)),
            pl.BlockSpec((_D, _OUT), lambda i: (0, 0)),
            pl.BlockSpec((1, _OUT), lambda i: (0, 0)),
        ],
        out_specs=[
            pl.BlockSpec((_BLK, _D), lambda i: (i, 0)),
            pl.BlockSpec((_BLK, _OUT), lambda i: (i, 0)),
        ],
        out_shape=[
            jax.ShapeDtypeStruct((_N, _D), jnp.float32),
            jax.ShapeDtypeStruct((_N, _OUT), jnp.float32),
        ],
    )(acc, acc, dinv_b, b3, pW1, pb1, pW2, pb2)


def kernel(x, edge_index, W1, b1, W2, b2, W3, b3, g1, be1, g2, be2,
           pW1, pb1, pW2, pb2):
    loop = jnp.arange(_N, dtype=jnp.int32)
    src = jnp.concatenate([edge_index[0].astype(jnp.int32), loop])
    dst = jnp.concatenate([edge_index[1].astype(jnp.int32), loop])
    npad = _EP - dst.shape[0]
    dstsf = jnp.concatenate(
        [dst, jnp.full((npad,), _N, jnp.int32)]).reshape(_NW, _STEPS * _CHUNK)
    apad = _AEP - src.shape[0]
    src_a = jnp.concatenate([src, jnp.zeros((apad,), jnp.int32)])
    dst_a = jnp.concatenate([dst, jnp.full((apad,), _N, jnp.int32)])
    srcs3 = src_a.reshape(-1, _NW).T.reshape(_NW, _ASTEPS, _ACHUNK)
    dsts3 = dst_a.reshape(-1, _NW).T.reshape(_NW, _ASTEPS, _ACHUNK)

    zeros_deg = jnp.zeros((_NP,), jnp.float32)
    zeros_acc = jnp.zeros((_NPA, _D), jnp.float32)
    b1r = b1.reshape(1, _D)
    b2r = b2.reshape(1, _D)
    b3r = b3.reshape(1, _D)
    g1r = g1.reshape(1, _D)
    be1r = be1.reshape(1, _D)
    g2r = g2.reshape(1, _D)
    be2r = be2.reshape(1, _D)
    pb1r = pb1.reshape(1, _D)
    pb2r = pb2.reshape(1, _OUT)

    deg2 = _deg_sc(dstsf, zeros_deg).reshape(_NW, _NP, 1)
    dinv_b = _tc_dinv(deg2)
    z0 = _tc_first(x, W1, dinv_b)
    a0 = _agg_sc(z0, srcs3, dsts3, zeros_acc).reshape(_NC, _NPA, _D)
    z1 = _tc_mid(a0, dinv_b, b1r, g1r, be1r, W2)
    a1 = _agg_sc(z1, srcs3, dsts3, zeros_acc).reshape(_NC, _NPA, _D)
    z2 = _tc_mid(a1, dinv_b, b2r, g2r, be2r, W3)
    a2 = _agg_sc(z2, srcs3, dsts3, zeros_acc).reshape(_NC, _NPA, _D)
    emb, out = _tc_last(a2, dinv_b, b3r, pW1, pb1r, pW2, pb2r)
    return emb, out
